# baseline (speedup 1.0000x reference)
import jax
import jax.numpy as jnp
from jax import lax
from jax.experimental import pallas as pl
from jax.experimental.pallas import tpu as pltpu

FEATS = 40
N_HIDDEN = FEATS * FEATS
GRU_IN = (FEATS + 1) * FEATS * 3
RB = 80
GRID = N_HIDDEN // RB


def _gat_star(dm, W, al, b):
    h = lax.dot_general(dm, W, (((1,), (1,)), ((), ())),
                        preferred_element_type=jnp.float32)
    el = h * al
    e = jnp.where(el > 0, el, 0.2 * el)
    emax = jnp.maximum(jnp.max(e, axis=0, keepdims=True), 0.0)
    ex = jnp.exp(e - emax)
    den = jnp.sum(ex, axis=0, keepdims=True) + jnp.exp(-emax)
    num = jnp.sum(ex * h, axis=0, keepdims=True)
    return num / den + b, h + b


def _body(d_ref, dT_ref, wf_ref, wt_ref, alf_ref, alt_ref, bf_ref, bt_ref,
          h0r_ref, h0m_ref, wih_ref, whh_ref, bihm_ref, bhhm_ref,
          out_ref, x_s, r_s, z_s):
    j = pl.program_id(0)

    @pl.when(j == 0)
    def _init():
        r0f, rrf = _gat_star(d_ref[...], wf_ref[...], alf_ref[...], bf_ref[...])
        r0t, rrt = _gat_star(dT_ref[...], wt_ref[...], alt_ref[...], bt_ref[...])
        fidx = lax.broadcasted_iota(jnp.int32, (FEATS, 3 * FEATS), 0)
        jidx = lax.broadcasted_iota(jnp.int32, (FEATS, 3 * FEATS), 1)
        P0 = (jidx == 3 * fidx).astype(jnp.float32)
        P1 = (jidx == 3 * fidx + 1).astype(jnp.float32)
        P2 = (jidx == 3 * fidx + 2).astype(jnp.float32)
        row0 = jnp.dot(r0f, P1, preferred_element_type=jnp.float32) + \
               jnp.dot(r0t, P2, preferred_element_type=jnp.float32)
        rows = jnp.dot(d_ref[...], P0, preferred_element_type=jnp.float32) + \
               jnp.dot(rrf, P1, preferred_element_type=jnp.float32) + \
               jnp.dot(rrt, P2, preferred_element_type=jnp.float32)
        W3 = 3 * FEATS
        x_s[:, 0:W3] = row0
        for n in range(FEATS):
            x_s[:, W3 * (n + 1):W3 * (n + 2)] = rows[n:n + 1, :]

    k = lax.rem(j, GRID)
    cdims = (((1,), (1,)), ((), ()))
    gi = lax.dot_general(x_s[...], wih_ref[...], cdims,
                         preferred_element_type=jnp.float32)
    gh = lax.dot_general(h0r_ref[...], whh_ref[...], cdims,
                         preferred_element_type=jnp.float32)
    bi = bihm_ref[pl.ds(j, 1), :]
    bh = bhhm_ref[pl.ds(j, 1), :]

    @pl.when(j < GRID)
    def _reset_gate():
        r_s[pl.ds(k, 1), :] = jax.nn.sigmoid(gi + gh + bi + bh)

    @pl.when(jnp.logical_and(j >= GRID, j < 2 * GRID))
    def _update_gate():
        z_s[pl.ds(k, 1), :] = jax.nn.sigmoid(gi + gh + bi + bh)

    @pl.when(j >= 2 * GRID)
    def _new_gate():
        r = r_s[pl.ds(k, 1), :]
        z = z_s[pl.ds(k, 1), :]
        n = jnp.tanh(gi + bi + r * (gh + bh))
        h0b = h0m_ref[pl.ds(k, 1), :]
        out_ref[pl.ds(k, 1), :] = (1.0 - z) * n + z * h0b


def kernel(data, hidden, W_feat, attn_l_feat, attn_r_feat, bias_feat,
           W_time, attn_l_time, attn_r_time, bias_time,
           W_ih, W_hh, b_ih, b_hh):
    del attn_r_feat, attn_r_time
    d = data.reshape(FEATS, FEATS)
    dT = d.T
    alf = attn_l_feat.reshape(1, FEATS)
    alt = attn_l_time.reshape(1, FEATS)
    bf = bias_feat.reshape(1, FEATS)
    bt = bias_time.reshape(1, FEATS)
    h0row = hidden.reshape(1, N_HIDDEN)
    h0m = hidden.reshape(GRID, RB)
    bihm = b_ih.reshape(3 * GRID, RB)
    bhhm = b_hh.reshape(3 * GRID, RB)

    full = lambda *s: pl.BlockSpec(s, lambda j: (0,) * len(s))
    hn = pl.pallas_call(
        _body,
        grid=(3 * GRID,),
        in_specs=[
            full(FEATS, FEATS),
            full(FEATS, FEATS),
            full(FEATS, FEATS),
            full(FEATS, FEATS),
            full(1, FEATS), full(1, FEATS),
            full(1, FEATS), full(1, FEATS),
            full(1, N_HIDDEN),
            full(GRID, RB),
            pl.BlockSpec((RB, GRU_IN), lambda j: (j, 0)),
            pl.BlockSpec((RB, N_HIDDEN), lambda j: (j, 0)),
            full(3 * GRID, RB),
            full(3 * GRID, RB),
        ],
        out_specs=pl.BlockSpec((GRID, RB), lambda j: (0, 0)),
        out_shape=jax.ShapeDtypeStruct((GRID, RB), jnp.float32),
        scratch_shapes=[pltpu.VMEM((1, GRU_IN), jnp.float32),
                        pltpu.VMEM((GRID, RB), jnp.float32),
                        pltpu.VMEM((GRID, RB), jnp.float32)],
        compiler_params=pltpu.CompilerParams(
            dimension_semantics=("arbitrary",)),
    )(d, dT, W_feat, W_time, alf, alt, bf, bt, h0row, h0m,
      W_ih, W_hh, bihm, bhhm)

    hn_flat = hn.reshape(N_HIDDEN)
    return hn_flat, hn_flat.reshape(1, 1, N_HIDDEN)
